# Initial kernel scaffold; baseline (speedup 1.0000x reference)
import jax
import jax.numpy as jnp
from jax.experimental import pallas as pl


def kernel(center, context, neg, W_center, W_context):
    raise NotImplementedError("write your pallas kernel here")


# speedup vs baseline: 3.9246x; 3.9246x over previous
import functools

import jax
import jax.numpy as jnp
from jax import lax
from jax.experimental import pallas as pl
from jax.experimental.pallas import tpu as pltpu
from jax.experimental.pallas import tpu_sc as plsc

D = 64
NEG = 20
NC = 2
NS = 16
NW = NC * NS
CE = 32
IDX_CHUNK = 128


def _sc_scores(center, context, neg_flat, W_center, W_context, B):
    PB = B // NW
    NCH = PB // CE
    mesh = plsc.VectorSubcoreMesh(core_axis_name="c", subcore_axis_name="s")

    @functools.partial(
        pl.kernel,
        out_type=(
            jax.ShapeDtypeStruct((B,), jnp.float32),
            jax.ShapeDtypeStruct((NEG, B), jnp.float32),
        ),
        mesh=mesh,
        compiler_params=pltpu.CompilerParams(needs_layout_passes=False,
                                             use_tc_tiling_on_sc=False),
        scratch_types=[
            pltpu.VMEM((PB,), jnp.int32),
            pltpu.VMEM((PB,), jnp.int32),
            pltpu.VMEM((PB * NEG,), jnp.int32),
            pltpu.VMEM((CE, D), jnp.float32),
            pltpu.VMEM((CE, D), jnp.float32),
            pltpu.VMEM((CE * NEG, D), jnp.float32),
            pltpu.VMEM((PB,), jnp.float32),
            pltpu.VMEM((NEG, PB), jnp.float32),
            pltpu.SemaphoreType.DMA,
        ],
    )
    def scores_kernel(center_h, context_h, neg_h, wc_h, wx_h,
                      pos_h, negt_h,
                      cidx_v, xidx_v, nidx_v, c_rows, x_rows, n_rows,
                      pos_v, negt_v, sem):
        wid = lax.axis_index("s") * NC + lax.axis_index("c")
        base = wid * PB

        pltpu.sync_copy(center_h.at[pl.ds(base, PB)], cidx_v)
        pltpu.sync_copy(context_h.at[pl.ds(base, PB)], xidx_v)
        pltpu.sync_copy(neg_h.at[pl.ds(base * NEG, PB * NEG)], nidx_v)

        lane = lax.iota(jnp.int32, 16)
        zero = jnp.zeros((16,), jnp.float32)

        for j in range(NCH):
            cps = [
                pltpu.async_copy(wc_h.at[cidx_v.at[pl.ds(j * CE, CE)]],
                                 c_rows, sem),
                pltpu.async_copy(wx_h.at[xidx_v.at[pl.ds(j * CE, CE)]],
                                 x_rows, sem),
            ]
            for i in range(CE * NEG // IDX_CHUNK):
                cps.append(pltpu.async_copy(
                    wx_h.at[nidx_v.at[pl.ds(j * CE * NEG + i * IDX_CHUNK,
                                            IDX_CHUNK)]],
                    n_rows.at[pl.ds(i * IDX_CHUNK, IDX_CHUNK)], sem))
            for cp in cps:
                cp.wait()

            for g in range(CE // 16):
                rowg = g * 16 + lane
                nrow = tuple(rowg * NEG + k for k in range(NEG))

                def dbody(d, carry):
                    col = jnp.full((16,), d, jnp.int32)
                    cT = plsc.load_gather(c_rows, [rowg, col])
                    xT = plsc.load_gather(x_rows, [rowg, col])
                    acc = (carry[0] + cT * xT,) + tuple(
                        carry[1 + k]
                        + cT * plsc.load_gather(n_rows, [nrow[k], col])
                        for k in range(NEG))
                    return acc

                acc = lax.fori_loop(0, D, dbody, (zero,) * (1 + NEG))
                off = j * CE + g * 16
                pos_v[pl.ds(off, 16)] = acc[0]
                for k in range(NEG):
                    negt_v[k, pl.ds(off, 16)] = acc[1 + k]

        pltpu.sync_copy(pos_v, pos_h.at[pl.ds(base, PB)])
        for k in range(NEG):
            pltpu.sync_copy(negt_v.at[k], negt_h.at[k, pl.ds(base, PB)])

    return scores_kernel(center, context, neg_flat, W_center, W_context)


def _loss_tc(pos2d, negt2d, B):
    def body(pos_ref, neg_ref, out_ref):
        p = pos_ref[...]
        n = neg_ref[...]
        pos_loss = -jnp.log(jax.nn.sigmoid(p) + 1e-8)
        neg_loss = -jnp.log(jax.nn.sigmoid(-n) + 1e-8)
        out_ref[0, 0] = (jnp.sum(pos_loss) + jnp.sum(neg_loss)) / B

    return pl.pallas_call(
        body,
        out_shape=jax.ShapeDtypeStruct((1, 1), jnp.float32),
        out_specs=pl.BlockSpec(memory_space=pltpu.SMEM),
    )(pos2d, negt2d)[0, 0]


def kernel(center, context, neg, W_center, W_context):
    B = center.shape[0]
    center = center.astype(jnp.int32)
    context = context.astype(jnp.int32)
    neg_flat = neg.astype(jnp.int32).reshape(-1)
    pos, negt = _sc_scores(center, context, neg_flat, W_center, W_context, B)
    pos2d = pos.reshape(B // 128, 128)
    negt2d = negt.reshape(NEG * B // 128, 128)
    return _loss_tc(pos2d, negt2d, B)
